# baseline (speedup 1.0000x reference)
import jax
import jax.numpy as jnp
from jax.experimental import pallas as pl
from jax.experimental.pallas import tpu as pltpu

_B, _C, _H, _W = 16, 96, 224, 224


def _body(x_hbm, o_hbm, const_ref, cp_sem, fill_sem):
    const_ref[...] = jnp.full((2, _H, _W), 4.0, jnp.float32)
    copies = []
    for b in range(_B):
        cp = pltpu.make_async_copy(
            x_hbm.at[b, pl.ds(2, _C - 2)], o_hbm.at[b, pl.ds(2, _C - 2)], cp_sem
        )
        cp.start()
        copies.append(cp)
        fl = pltpu.make_async_copy(const_ref, o_hbm.at[b, pl.ds(0, 2)], fill_sem)
        fl.start()
        copies.append(fl)
    for cp in copies:
        cp.wait()


def kernel(x):
    return pl.pallas_call(
        _body,
        in_specs=[pl.BlockSpec(memory_space=pl.ANY)],
        out_specs=pl.BlockSpec(memory_space=pl.ANY),
        out_shape=jax.ShapeDtypeStruct((_B, _C, _H, _W), x.dtype),
        scratch_shapes=[
            pltpu.VMEM((2, _H, _W), jnp.float32),
            pltpu.SemaphoreType.DMA,
            pltpu.SemaphoreType.DMA,
        ],
    )(x)


# speedup vs baseline: 47.8666x; 47.8666x over previous
import jax
import jax.numpy as jnp
from jax.experimental import pallas as pl

_B, _C, _H, _W = 16, 96, 224, 224
_CB = 32


def _body(x_ref, o_ref):
    j = pl.program_id(1)

    @pl.when(j == 0)
    def _():
        c = jax.lax.broadcasted_iota(jnp.int32, (1, _CB, 1, 1), 1)
        o_ref[...] = jnp.where(c < 2, jnp.float32(4.0), x_ref[...])

    @pl.when(j != 0)
    def _():
        o_ref[...] = x_ref[...]


def kernel(x):
    return pl.pallas_call(
        _body,
        grid=(_B, _C // _CB),
        in_specs=[pl.BlockSpec((1, _CB, _H, _W), lambda i, j: (i, j, 0, 0))],
        out_specs=pl.BlockSpec((1, _CB, _H, _W), lambda i, j: (i, j, 0, 0)),
        out_shape=jax.ShapeDtypeStruct((_B, _C, _H, _W), x.dtype),
    )(x)


# speedup vs baseline: 48.1191x; 1.0053x over previous
import jax
import jax.numpy as jnp
from jax.experimental import pallas as pl

_B, _C, _H, _W = 16, 96, 224, 224
_CB = 48


def _body(x_ref, o_ref):
    j = pl.program_id(1)

    @pl.when(j == 0)
    def _():
        c = jax.lax.broadcasted_iota(jnp.int32, (1, _CB, 1, 1), 1)
        o_ref[...] = jnp.where(c < 2, jnp.float32(4.0), x_ref[...])

    @pl.when(j != 0)
    def _():
        o_ref[...] = x_ref[...]


def kernel(x):
    return pl.pallas_call(
        _body,
        grid=(_B, _C // _CB),
        in_specs=[pl.BlockSpec((1, _CB, _H, _W), lambda i, j: (i, j, 0, 0))],
        out_specs=pl.BlockSpec((1, _CB, _H, _W), lambda i, j: (i, j, 0, 0)),
        out_shape=jax.ShapeDtypeStruct((_B, _C, _H, _W), x.dtype),
    )(x)


# speedup vs baseline: 48.1520x; 1.0007x over previous
import jax
import jax.numpy as jnp
from jax.experimental import pallas as pl
from jax.experimental.pallas import tpu as pltpu

_B, _C, _H, _W = 16, 96, 224, 224
_CB = 48


def _body(x_ref, o_ref):
    j = pl.program_id(1)

    @pl.when(j == 0)
    def _():
        c = jax.lax.broadcasted_iota(jnp.int32, (1, _CB, 1, 1), 1)
        o_ref[...] = jnp.where(c < 2, jnp.float32(4.0), x_ref[...])

    @pl.when(j != 0)
    def _():
        o_ref[...] = x_ref[...]


def kernel(x):
    return pl.pallas_call(
        _body,
        grid=(_B, _C // _CB),
        in_specs=[pl.BlockSpec((1, _CB, _H, _W), lambda i, j: (i, j, 0, 0))],
        out_specs=pl.BlockSpec((1, _CB, _H, _W), lambda i, j: (i, j, 0, 0)),
        out_shape=jax.ShapeDtypeStruct((_B, _C, _H, _W), x.dtype),
        compiler_params=pltpu.CompilerParams(
            dimension_semantics=("parallel", "parallel"),
        ),
    )(x)
